# baseline (speedup 1.0000x reference)
import functools

import jax
import jax.numpy as jnp
from jax import lax
from jax.experimental import pallas as pl
from jax.experimental.pallas import tpu as pltpu
from jax.experimental.pallas import tpu_sc as plsc

_NC = 2
_NS = 16
_NW = _NC * _NS
_R = 16
_LW = 128
_BM = 64


def _gather_flat(x, C):
    B, H = x.shape
    V, D = C.shape
    rpw = B // _NW
    nch = rpw // _R
    ppc = _R * H * D // _LW
    assert rpw * _NW == B and nch * _R == rpw and ppc * _LW == _R * H * D
    assert nch >= 6 and nch % 2 == 0

    mesh = plsc.VectorSubcoreMesh(
        core_axis_name="c", subcore_axis_name="s",
        num_cores=_NC, num_subcores=_NS)

    @functools.partial(
        pl.kernel,
        out_type=jax.ShapeDtypeStruct((B * H, D), jnp.float32),
        mesh=mesh,
        scratch_types=[
            pltpu.VMEM((rpw, H), jnp.int32),
            pltpu.VMEM((2, _R * H, D), jnp.float32),
            pltpu.SemaphoreType.DMA,
            pltpu.SemaphoreType.DMA,
            pltpu.SemaphoreType.DMA,
            pltpu.SemaphoreType.DMA,
        ],
        compiler_params=pltpu.CompilerParams(use_tc_tiling_on_sc=False),
    )
    def emb(x_hbm, C_hbm, p_hbm, xslab, rows_v, g0, g1, o0, o1):
        wid = lax.axis_index("s") * _NC + lax.axis_index("c")
        row0 = wid * rpw
        p0 = wid * rpw * H
        npc = _R * H
        gsem = (g0, g1)
        osem = (o0, o1)

        pltpu.sync_copy(x_hbm.at[pl.ds(row0, rpw)], xslab)

        def gfire(c, s):
            for r in range(_R):
                pltpu.async_copy(C_hbm.at[xslab.at[c * _R + r]],
                                 rows_v.at[s, pl.ds(r * H, H)], gsem[s])

        def gdrain(s):
            pltpu.make_async_copy(
                p_hbm.at[pl.ds(0, npc)], rows_v.at[s], gsem[s]).wait()

        def sfire(c, s):
            pltpu.async_copy(rows_v.at[s],
                             p_hbm.at[pl.ds(p0 + c * npc, npc)], osem[s])

        def sdrain(s):
            pltpu.make_async_copy(
                rows_v.at[s], p_hbm.at[pl.ds(0, npc)], osem[s]).wait()

        gfire(0, 0)
        gfire(1, 1)
        gdrain(0)
        sfire(0, 0)
        sdrain(0)
        gfire(2, 0)
        gdrain(1)
        sfire(1, 1)

        @pl.loop(3, nch - 2, step=2)
        def pair(c):
            sdrain(1)
            gfire(c, 1)
            gdrain(0)
            sfire(c - 1, 0)
            sdrain(0)
            gfire(c + 1, 0)
            gdrain(1)
            sfire(c, 1)

        sdrain(1)
        gfire(nch - 1, 1)
        gdrain(0)
        sfire(nch - 2, 0)
        gdrain(1)
        sfire(nch - 1, 1)
        sdrain(0)
        sdrain(1)

    return emb(x, C)


def _unflatten_tc(p, B, H, D):
    rpb = _BM * H * D // _LW

    nq = _LW // D

    def body(p_ref, o_ref):
        pb = p_ref[...]
        cols = jnp.stack([pb[:, q * D:(q + 1) * D] for q in range(nq)],
                         axis=1)
        o_ref[...] = cols.reshape(_BM, H, D)

    return pl.pallas_call(
        body,
        grid=(B // _BM,),
        in_specs=[pl.BlockSpec((rpb, _LW), lambda i: (i, 0))],
        out_specs=pl.BlockSpec((_BM, H, D), lambda i: (i, 0, 0)),
        out_shape=jax.ShapeDtypeStruct((B, H, D), jnp.float32),
    )(p)


def kernel(x, C):
    B, H = x.shape
    V, D = C.shape
    p = _gather_flat(x, C)
    return _unflatten_tc(p.reshape(B * H * D // _LW, _LW), B, H, D)


# speedup vs baseline: 1.5025x; 1.5025x over previous
import functools

import jax
import jax.numpy as jnp
from jax import lax
from jax.experimental import pallas as pl
from jax.experimental.pallas import tpu as pltpu
from jax.experimental.pallas import tpu_sc as plsc

_NC = 2
_NS = 16
_NW = _NC * _NS
_R = 16


def kernel(x, C):
    B, H = x.shape
    V, D = C.shape
    rpw = B // _NW
    nch = rpw // _R
    assert rpw * _NW == B and nch * _R == rpw
    assert nch >= 6 and nch % 2 == 0

    mesh = plsc.VectorSubcoreMesh(
        core_axis_name="c", subcore_axis_name="s",
        num_cores=_NC, num_subcores=_NS)

    @functools.partial(
        pl.kernel,
        out_type=jax.ShapeDtypeStruct((B, H, D), jnp.float32),
        mesh=mesh,
        scratch_types=[
            pltpu.VMEM((rpw, H), jnp.int32),
            pltpu.VMEM((2, _R, H, D), jnp.float32),
            pltpu.SemaphoreType.DMA,
            pltpu.SemaphoreType.DMA,
            pltpu.SemaphoreType.DMA,
            pltpu.SemaphoreType.DMA,
        ],
        compiler_params=pltpu.CompilerParams(use_tc_tiling_on_sc=False),
    )
    def emb(x_hbm, C_hbm, out_hbm, xslab, rows_v, g0, g1, o0, o1):
        wid = lax.axis_index("s") * _NC + lax.axis_index("c")
        row0 = wid * rpw
        gsem = (g0, g1)
        osem = (o0, o1)

        pltpu.sync_copy(x_hbm.at[pl.ds(row0, rpw)], xslab)

        def gfire(c, s):
            for r in range(_R):
                pltpu.async_copy(C_hbm.at[xslab.at[c * _R + r]],
                                 rows_v.at[s, r], gsem[s])

        def gdrain(s):
            pltpu.make_async_copy(
                out_hbm.at[pl.ds(0, _R)], rows_v.at[s], gsem[s]).wait()

        def sfire(c, s):
            pltpu.async_copy(rows_v.at[s],
                             out_hbm.at[pl.ds(row0 + c * _R, _R)], osem[s])

        def sdrain(s):
            pltpu.make_async_copy(
                rows_v.at[s], out_hbm.at[pl.ds(0, _R)], osem[s]).wait()

        gfire(0, 0)
        gfire(1, 1)
        gdrain(0)
        sfire(0, 0)
        sdrain(0)
        gfire(2, 0)
        gdrain(1)
        sfire(1, 1)

        @pl.loop(3, nch - 2, step=2)
        def pair(c):
            sdrain(1)
            gfire(c, 1)
            gdrain(0)
            sfire(c - 1, 0)
            sdrain(0)
            gfire(c + 1, 0)
            gdrain(1)
            sfire(c, 1)

        sdrain(1)
        gfire(nch - 1, 1)
        gdrain(0)
        sfire(nch - 2, 0)
        gdrain(1)
        sfire(nch - 1, 1)
        sdrain(0)
        sdrain(1)

    return emb(x, C)
